# Initial kernel scaffold; baseline (speedup 1.0000x reference)
import jax
import jax.numpy as jnp
from jax.experimental import pallas as pl


def kernel(x, edge_index, edge_attr, batch, ne_w1, ne_b1, ne_w2, ne_b2, ne_g, ne_be, ee_w1, ee_b1, ee_w2, ee_b2, ee_g, ee_be, c1_w, c1_b, c2_w, c2_b, ctx_w, ctx_b, ctx_g, ctx_be, s1_w, s1_b, s2_w, s2_b, s3_w, s3_b):
    raise NotImplementedError("write your pallas kernel here")


# speedup vs baseline: 3.4858x; 3.4858x over previous
import functools

import jax
import jax.numpy as jnp
from jax import lax
from jax.experimental import pallas as pl
from jax.experimental.pallas import tpu as pltpu
from jax.experimental.pallas import tpu_sc as plsc

N = 10000
E = 160000
H = 256
G = 8
NC = 2
NS = 16
NPAD = 10240
EPAD = 163840
KB = 128
BN = 1000
BE = 1280

_f32 = jnp.float32


def _sc_mesh():
    return plsc.VectorSubcoreMesh(core_axis_name="c", subcore_axis_name="s")


def _sc_degree(dst_pad, ones16, zeros_hist):
    epw = EPAD // (NC * NS)
    nblk = epw // KB

    @functools.partial(
        pl.kernel,
        mesh=_sc_mesh(),
        out_type=[jax.ShapeDtypeStruct((NPAD, 128), _f32)] * 2,
        scratch_types=[
            pltpu.VMEM((1, KB), jnp.int32),
            pltpu.VMEM((KB, 128), _f32),
            pltpu.VMEM_SHARED((NPAD, 128), _f32),
        ],
    )
    def k(dst_hbm, ones_hbm, zz_hbm, da_hbm, db_hbm, didx, ones_v, hist):
        c = lax.axis_index("c")
        s = lax.axis_index("s")
        rpw = NPAD // NS
        pltpu.sync_copy(zz_hbm.at[pl.ds(s * rpw, rpw)], hist.at[pl.ds(s * rpw, rpw)])
        pltpu.sync_copy(ones_hbm, ones_v)
        plsc.subcore_barrier()
        base = (s * NC + c) * epw

        @pl.loop(0, nblk)
        def _(i):
            pltpu.sync_copy(dst_hbm.at[pl.ds(base + i * KB, KB)], didx.at[0])
            pltpu.sync_copy(ones_v, hist.at[didx.at[0]], add=True)

        plsc.subcore_barrier()

        @pl.when(c == 0)
        def _():
            pltpu.sync_copy(hist.at[pl.ds(s * rpw, rpw)], da_hbm.at[pl.ds(s * rpw, rpw)])

        @pl.when(c == 1)
        def _():
            pltpu.sync_copy(hist.at[pl.ds(s * rpw, rpw)], db_hbm.at[pl.ds(s * rpw, rpw)])

    return k(dst_pad, ones16, zeros_hist)


def _sc_conv(ua, ub, src_pad, dst_pad, zeros_big):
    eps = EPAD // NS
    nblk = eps // KB

    @functools.partial(
        pl.kernel,
        mesh=_sc_mesh(),
        out_type=[jax.ShapeDtypeStruct((NPAD, 128), _f32)] * 2,
        scratch_types=[
            pltpu.VMEM((KB,), jnp.int32),
            pltpu.VMEM((KB,), jnp.int32),
            pltpu.VMEM((KB, 128), _f32),
            pltpu.VMEM_SHARED((NPAD, 128), _f32),
            pltpu.SemaphoreType.DMA,
        ],
    )
    def k(ua_hbm, ub_hbm, src_hbm, dst_hbm, zz_hbm, ta_hbm, tb_hbm,
          sidx, didx, rows, acc, sem):
        c = lax.axis_index("c")
        s = lax.axis_index("s")
        rpw = NPAD // NS
        pltpu.sync_copy(zz_hbm.at[pl.ds(s * rpw, rpw)], acc.at[pl.ds(s * rpw, rpw)])
        plsc.subcore_barrier()
        base = s * eps

        @pl.loop(0, nblk)
        def _(i):
            b = base + i * KB
            pltpu.sync_copy(src_hbm.at[pl.ds(b, KB)], sidx)
            pltpu.sync_copy(dst_hbm.at[pl.ds(b, KB)], didx)

            @pl.when(c == 0)
            def _():
                pltpu.async_copy(ua_hbm.at[sidx], rows, sem).wait()

            @pl.when(c == 1)
            def _():
                pltpu.async_copy(ub_hbm.at[sidx], rows, sem).wait()

            pltpu.sync_copy(rows, acc.at[didx], add=True)

        plsc.subcore_barrier()

        @pl.when(c == 0)
        def _():
            pltpu.sync_copy(acc.at[pl.ds(s * rpw, rpw)], ta_hbm.at[pl.ds(s * rpw, rpw)])

        @pl.when(c == 1)
        def _():
            pltpu.sync_copy(acc.at[pl.ds(s * rpw, rpw)], tb_hbm.at[pl.ds(s * rpw, rpw)])

    return k(ua, ub, src_pad, dst_pad, zeros_big)


def _sc_zgather(a2, bta, src_pad, dst0_pad):
    KZ = 64
    epw = EPAD // (NC * NS)
    nblk = epw // KZ

    @functools.partial(
        pl.kernel,
        mesh=_sc_mesh(),
        out_type=[jax.ShapeDtypeStruct((EPAD, 512), _f32)] * 2,
        scratch_types=[
            pltpu.VMEM((KZ,), jnp.int32),
            pltpu.VMEM((KZ,), jnp.int32),
            pltpu.VMEM((KZ, 512), _f32),
            pltpu.VMEM((KZ, 512), _f32),
            pltpu.SemaphoreType.DMA,
            pltpu.SemaphoreType.DMA,
        ],
    )
    def k(a_hbm, b_hbm, src_hbm, dst_hbm, za_hbm, zb_hbm,
          sidx, didx, rowsa, rowsb, sema, semb):
        c = lax.axis_index("c")
        s = lax.axis_index("s")
        base = (s * NC + c) * epw

        @pl.loop(0, nblk)
        def _(i):
            b = base + i * KZ
            pltpu.sync_copy(src_hbm.at[pl.ds(b, KZ)], sidx)
            pltpu.sync_copy(dst_hbm.at[pl.ds(b, KZ)], didx)
            cpa = pltpu.async_copy(a_hbm.at[sidx], rowsa, sema)
            cpb = pltpu.async_copy(b_hbm.at[didx], rowsb, semb)
            cpa.wait()
            pltpu.sync_copy(rowsa, za_hbm.at[pl.ds(b, KZ)])
            cpb.wait()
            pltpu.sync_copy(rowsb, zb_hbm.at[pl.ds(b, KZ)])

    return k(a2, bta, src_pad, dst0_pad)


def _row(v):
    return v.reshape(1, -1)


def _ln_rows(z, g, b):
    m = jnp.mean(z, axis=1, keepdims=True)
    v = jnp.mean((z - m) * (z - m), axis=1, keepdims=True)
    return (z - m) * lax.rsqrt(v + 1e-5) * g + b


def _tc_node_enc(x, w1, b1, w2, b2, g, be):
    def body(x_r, w1_r, b1_r, w2_r, b2_r, g_r, be_r, o_r):
        a = jnp.maximum(jnp.dot(x_r[...], w1_r[...],
                                preferred_element_type=_f32) + b1_r[...], 0.0)
        z = jnp.dot(a, w2_r[...], preferred_element_type=_f32) + b2_r[...]
        o_r[...] = _ln_rows(z, g_r[...], be_r[...])

    return pl.pallas_call(
        body,
        grid=(N // BN,),
        in_specs=[
            pl.BlockSpec((BN, H), lambda i: (i, 0)),
            pl.BlockSpec((H, H), lambda i: (0, 0)),
            pl.BlockSpec((1, H), lambda i: (0, 0)),
            pl.BlockSpec((H, H), lambda i: (0, 0)),
            pl.BlockSpec((1, H), lambda i: (0, 0)),
            pl.BlockSpec((1, H), lambda i: (0, 0)),
            pl.BlockSpec((1, H), lambda i: (0, 0)),
        ],
        out_specs=pl.BlockSpec((BN, H), lambda i: (i, 0)),
        out_shape=jax.ShapeDtypeStruct((N, H), _f32),
    )(x, w1, _row(b1), w2, _row(b2), _row(g), _row(be))


def _tc_dis_u(h, W, dega, degb):
    def body(h_r, w_r, da_r, db_r, dis_r, ua_r, ub_r):
        deg = da_r[...] + db_r[...] + 1.0
        dis = lax.rsqrt(deg)
        dis_r[...] = dis
        u = jnp.dot(h_r[...], w_r[...], preferred_element_type=_f32) * dis
        ua_r[...] = u[:, :128]
        ub_r[...] = u[:, 128:]

    return pl.pallas_call(
        body,
        grid=(N // BN,),
        in_specs=[
            pl.BlockSpec((BN, H), lambda i: (i, 0)),
            pl.BlockSpec((H, H), lambda i: (0, 0)),
            pl.BlockSpec((BN, 1), lambda i: (i, 0)),
            pl.BlockSpec((BN, 1), lambda i: (i, 0)),
        ],
        out_specs=[
            pl.BlockSpec((BN, 1), lambda i: (i, 0)),
            pl.BlockSpec((BN, 128), lambda i: (i, 0)),
            pl.BlockSpec((BN, 128), lambda i: (i, 0)),
        ],
        out_shape=[
            jax.ShapeDtypeStruct((N, 1), _f32),
            jax.ShapeDtypeStruct((N, 128), _f32),
            jax.ShapeDtypeStruct((N, 128), _f32),
        ],
    )(h, W, dega, degb)


def _tc_conv_next(ta, tb, ua, ub, dis, b, W):
    def body(ta_r, tb_r, ua_r, ub_r, dis_r, b_r, w_r, ua2_r, ub2_r):
        dis = dis_r[...]
        hz = jnp.concatenate([ta_r[...] + ua_r[...], tb_r[...] + ub_r[...]],
                             axis=1) * dis + b_r[...]
        h1 = jnp.where(hz > 0, hz, 0.1 * hz)
        u2 = jnp.dot(h1, w_r[...], preferred_element_type=_f32) * dis
        ua2_r[...] = u2[:, :128]
        ub2_r[...] = u2[:, 128:]

    return pl.pallas_call(
        body,
        grid=(N // BN,),
        in_specs=[
            pl.BlockSpec((BN, 128), lambda i: (i, 0)),
            pl.BlockSpec((BN, 128), lambda i: (i, 0)),
            pl.BlockSpec((BN, 128), lambda i: (i, 0)),
            pl.BlockSpec((BN, 128), lambda i: (i, 0)),
            pl.BlockSpec((BN, 1), lambda i: (i, 0)),
            pl.BlockSpec((1, H), lambda i: (0, 0)),
            pl.BlockSpec((H, H), lambda i: (0, 0)),
        ],
        out_specs=[
            pl.BlockSpec((BN, 128), lambda i: (i, 0)),
            pl.BlockSpec((BN, 128), lambda i: (i, 0)),
        ],
        out_shape=[
            jax.ShapeDtypeStruct((N, 128), _f32),
            jax.ShapeDtypeStruct((N, 128), _f32),
        ],
    )(ta, tb, ua, ub, dis, _row(b), W)


def _tc_h_gc(ta, tb, ua, ub, dis, b, batch):
    def body(ta_r, tb_r, ua_r, ub_r, dis_r, b_r, batch_r, h_r, gs_r, cnt_r):
        i = pl.program_id(0)
        h = jnp.concatenate([ta_r[...] + ua_r[...], tb_r[...] + ub_r[...]],
                            axis=1) * dis_r[...] + b_r[...]
        h_r[...] = h

        @pl.when(i == 0)
        def _():
            gs_r[...] = jnp.zeros_like(gs_r)
            cnt_r[...] = jnp.zeros_like(cnt_r)

        bt = batch_r[...]
        rows = []
        cnts = []
        for g in range(G):
            mask = (bt == g).astype(_f32)
            rows.append(jnp.sum(h * mask, axis=0, keepdims=True))
            cnts.append(jnp.sum(mask) * jnp.ones((1, 128), _f32))
        gs_r[...] += jnp.concatenate(rows, axis=0)
        cnt_r[...] += jnp.concatenate(cnts, axis=0)

    return pl.pallas_call(
        body,
        grid=(N // BN,),
        in_specs=[
            pl.BlockSpec((BN, 128), lambda i: (i, 0)),
            pl.BlockSpec((BN, 128), lambda i: (i, 0)),
            pl.BlockSpec((BN, 128), lambda i: (i, 0)),
            pl.BlockSpec((BN, 128), lambda i: (i, 0)),
            pl.BlockSpec((BN, 1), lambda i: (i, 0)),
            pl.BlockSpec((1, H), lambda i: (0, 0)),
            pl.BlockSpec((BN, 1), lambda i: (i, 0)),
        ],
        out_specs=[
            pl.BlockSpec((BN, H), lambda i: (i, 0)),
            pl.BlockSpec((G, H), lambda i: (0, 0)),
            pl.BlockSpec((G, 128), lambda i: (0, 0)),
        ],
        out_shape=[
            jax.ShapeDtypeStruct((N, H), _f32),
            jax.ShapeDtypeStruct((G, H), _f32),
            jax.ShapeDtypeStruct((G, 128), _f32),
        ],
    )(ta, tb, ua, ub, dis, _row(b), batch)


def _tc_ctx(gs, cnt, ctx_w, ctx_b, ctx_g, ctx_be, Wg, s1_b):
    def body(gs_r, cnt_r, w_r, b_r, g_r, be_r, wg_r, s1b_r, o_r):
        gc = gs_r[...] / jnp.maximum(cnt_r[:, 0:1], 1.0)
        a = jnp.maximum(jnp.dot(gc, w_r[...],
                                preferred_element_type=_f32) + b_r[...], 0.0)
        gc2 = _ln_rows(a, g_r[...], be_r[...])
        o_r[...] = jnp.dot(gc2, wg_r[...],
                           preferred_element_type=_f32) + s1b_r[...]

    return pl.pallas_call(
        body,
        grid=(1,),
        in_specs=[
            pl.BlockSpec((G, H), lambda i: (0, 0)),
            pl.BlockSpec((G, 128), lambda i: (0, 0)),
            pl.BlockSpec((H, H), lambda i: (0, 0)),
            pl.BlockSpec((1, H), lambda i: (0, 0)),
            pl.BlockSpec((1, H), lambda i: (0, 0)),
            pl.BlockSpec((1, H), lambda i: (0, 0)),
            pl.BlockSpec((H, 512), lambda i: (0, 0)),
            pl.BlockSpec((1, 512), lambda i: (0, 0)),
        ],
        out_specs=pl.BlockSpec((G, 512), lambda i: (0, 0)),
        out_shape=jax.ShapeDtypeStruct((G, 512), _f32),
    )(gs, cnt, ctx_w, _row(ctx_b), _row(ctx_g), _row(ctx_be), Wg, _row(s1_b))


def _tc_tables(h, batch, gcs, Wsd):
    def body(h_r, batch_r, gcs_r, w_r, a_r, b_r):
        ab = jnp.dot(h_r[...], w_r[...], preferred_element_type=_f32)
        a2 = ab[:, :512]
        bt = batch_r[...]
        for g in range(G):
            mask = (bt == g).astype(_f32)
            a2 = a2 + mask * gcs_r[g:g + 1, :]
        a_r[...] = a2
        b_r[...] = ab[:, 512:]

    return pl.pallas_call(
        body,
        grid=(N // BN,),
        in_specs=[
            pl.BlockSpec((BN, H), lambda i: (i, 0)),
            pl.BlockSpec((BN, 1), lambda i: (i, 0)),
            pl.BlockSpec((G, 512), lambda i: (0, 0)),
            pl.BlockSpec((H, 1024), lambda i: (0, 0)),
        ],
        out_specs=[
            pl.BlockSpec((BN, 512), lambda i: (i, 0)),
            pl.BlockSpec((BN, 512), lambda i: (i, 0)),
        ],
        out_shape=[
            jax.ShapeDtypeStruct((N, 512), _f32),
            jax.ShapeDtypeStruct((N, 512), _f32),
        ],
    )(h, batch, gcs, Wsd)


def _tc_scorer(za, zb, edge_attr, ee_w1, ee_b1, ee_w2, ee_b2, ee_g, ee_be,
               Wc, s2_w, s2_b, s3_w, s3_b):
    w2_bf = ee_w2.astype(jnp.bfloat16)
    wc_bf = Wc.astype(jnp.bfloat16)
    s2_bf = s2_w.astype(jnp.bfloat16)

    def body(za_r, zb_r, ea_r, w1_r, b1_r, w2_r, b2_r, g_r, be_r,
             wc_r, s2_r, s2b_r, s3_r, s3b_r, o_r):
        a = jnp.maximum(jnp.dot(ea_r[...], w1_r[...],
                                preferred_element_type=_f32) + b1_r[...], 0.0)
        y = jnp.dot(a.astype(jnp.bfloat16), w2_r[...],
                    preferred_element_type=_f32) + b2_r[...]
        he = _ln_rows(y, g_r[...], be_r[...])
        c = jnp.dot(he.astype(jnp.bfloat16), wc_r[...],
                    preferred_element_type=_f32)
        z = za_r[...] + zb_r[...] + c
        sct = jnp.tanh(z)
        t = jnp.tanh(jnp.dot(sct.astype(jnp.bfloat16), s2_r[...],
                             preferred_element_type=_f32) + s2b_r[...])
        o = jax.nn.sigmoid(jnp.dot(t, s3_r[...],
                                   preferred_element_type=_f32) + s3b_r[...])
        o_r[...] = o

    return pl.pallas_call(
        body,
        grid=(E // BE,),
        in_specs=[
            pl.BlockSpec((BE, 512), lambda i: (i, 0)),
            pl.BlockSpec((BE, 512), lambda i: (i, 0)),
            pl.BlockSpec((BE, 16), lambda i: (i, 0)),
            pl.BlockSpec((16, H), lambda i: (0, 0)),
            pl.BlockSpec((1, H), lambda i: (0, 0)),
            pl.BlockSpec((H, H), lambda i: (0, 0)),
            pl.BlockSpec((1, H), lambda i: (0, 0)),
            pl.BlockSpec((1, H), lambda i: (0, 0)),
            pl.BlockSpec((1, H), lambda i: (0, 0)),
            pl.BlockSpec((H, 512), lambda i: (0, 0)),
            pl.BlockSpec((512, H), lambda i: (0, 0)),
            pl.BlockSpec((1, H), lambda i: (0, 0)),
            pl.BlockSpec((H, 1), lambda i: (0, 0)),
            pl.BlockSpec((1, 1), lambda i: (0, 0)),
        ],
        out_specs=pl.BlockSpec((BE, 1), lambda i: (i, 0)),
        out_shape=jax.ShapeDtypeStruct((E, 1), _f32),
    )(za, zb, edge_attr, ee_w1, _row(ee_b1), w2_bf, _row(ee_b2), _row(ee_g),
      _row(ee_be), wc_bf, s2_bf, _row(s2_b), s3_w, s3_b.reshape(1, 1))


def kernel(x, edge_index, edge_attr, batch,
           ne_w1, ne_b1, ne_w2, ne_b2, ne_g, ne_be,
           ee_w1, ee_b1, ee_w2, ee_b2, ee_g, ee_be,
           c1_w, c1_b, c2_w, c2_b,
           ctx_w, ctx_b, ctx_g, ctx_be,
           s1_w, s1_b, s2_w, s2_b, s3_w, s3_b):
    src = edge_index[0].astype(jnp.int32)
    dst = edge_index[1].astype(jnp.int32)
    padn = EPAD - E
    src_pad = jnp.concatenate([src, jnp.zeros((padn,), jnp.int32)])
    dst_sc = jnp.concatenate([dst, jnp.full((padn,), N, jnp.int32)])
    dst0_pad = jnp.concatenate([dst, jnp.zeros((padn,), jnp.int32)])

    ones128 = jnp.ones((KB, 128), _f32)
    zeros_big = jnp.zeros((NPAD, 128), _f32)
    batch2 = batch.astype(jnp.int32).reshape(N, 1)

    dega, degb = _sc_degree(dst_sc, ones128, zeros_big)
    dega_c = dega[:N, 0:1]
    degb_c = degb[:N, 0:1]

    h_nodes = _tc_node_enc(x, ne_w1, ne_b1, ne_w2, ne_b2, ne_g, ne_be)
    dis, ua, ub = _tc_dis_u(h_nodes, c1_w, dega_c, degb_c)

    ta, tb = _sc_conv(ua, ub, src_pad, dst_sc, zeros_big)
    u2a, u2b = _tc_conv_next(ta, tb, ua, ub, dis, c1_b, c2_w)

    t2a, t2b = _sc_conv(u2a, u2b, src_pad, dst_sc, zeros_big)
    h, gs, cnt = _tc_h_gc(t2a, t2b, u2a, u2b, dis, c2_b, batch2)

    Wg = s1_w[768:1024]
    gcs = _tc_ctx(gs, cnt, ctx_w, ctx_b, ctx_g, ctx_be, Wg, s1_b)

    Wsd = jnp.concatenate([s1_w[0:256], s1_w[256:512]], axis=1)
    a2, bta = _tc_tables(h, batch2, gcs, Wsd)

    za, zb = _sc_zgather(a2, bta, src_pad, dst0_pad)

    Wc = s1_w[512:768]
    return _tc_scorer(za, zb, edge_attr, ee_w1, ee_b1, ee_w2, ee_b2, ee_g,
                      ee_be, Wc, s2_w, s2_b, s3_w, s3_b)
